# baseline (speedup 1.0000x reference)
import functools
import math

import jax
import jax.numpy as jnp
from jax import lax
from jax.experimental import pallas as pl
from jax.experimental.pallas import tpu as pltpu
from jax.experimental.pallas import tpu_sc as plsc

EMB = 32
SCALE = math.sqrt(32.0)

ROWS = 16384
COLS = 50
NW = 32
R_PER_W = ROWS // NW
TR = 16
N_CHUNKS = R_PER_W // TR
LOOK = TR * COLS
OUT_W = 128
PACK = OUT_W // EMB
OUT_ROWS = ROWS * COLS // PACK
OR_PER_CHUNK = LOOK // PACK

_mesh = plsc.VectorSubcoreMesh(core_axis_name="c", subcore_axis_name="s")


@functools.partial(
    pl.kernel,
    mesh=_mesh,
    out_type=jax.ShapeDtypeStruct((OUT_ROWS, OUT_W), jnp.float32),
    scratch_types=[
        pltpu.VMEM((2, TR, COLS), jnp.int32),
        pltpu.VMEM((2, LOOK, EMB), jnp.float32),
        pltpu.VMEM((2, OR_PER_CHUNK, OUT_W), jnp.float32),
        pltpu.SemaphoreType.DMA((2,)),
    ],
    compiler_params=pltpu.CompilerParams(use_tc_tiling_on_sc=False),
)
def _emb_lookup(tok_hbm, table_hbm, out_hbm, idx_v, rows_v, out_v, gsem):
    wid = lax.axis_index("s") * 2 + lax.axis_index("c")
    base = wid * R_PER_W

    def fire(ci, b):
        r0 = base + ci * TR
        pltpu.sync_copy(tok_hbm.at[pl.ds(r0, TR)], idx_v.at[b])
        for r in range(TR):
            pltpu.async_copy(
                table_hbm.at[idx_v.at[b, r]],
                rows_v.at[b, pl.ds(r * COLS, COLS)],
                gsem.at[b],
            )

    def drain(b):
        for r in range(TR):
            pltpu.make_async_copy(
                table_hbm.at[idx_v.at[b, r]],
                rows_v.at[b, pl.ds(r * COLS, COLS)],
                gsem.at[b],
            ).wait()

    fire(0, 0)

    def outer(g, carry):
        for b in (0, 1):
            ci = g * 2 + b
            nb = 1 - b

            @pl.when(ci + 1 < N_CHUNKS)
            def _():
                fire(ci + 1, nb)

            drain(b)

            @plsc.parallel_loop(0, LOOK, step=1, unroll=8)
            def _(q):
                fr = q // PACK
                c0 = (q % PACK) * EMB
                out_v[b, fr, pl.ds(c0, 16)] = rows_v[b, q, pl.ds(0, 16)]
                out_v[b, fr, pl.ds(c0 + 16, 16)] = rows_v[b, q, pl.ds(16, 16)]

            f0 = wid * (R_PER_W * COLS // PACK) + ci * OR_PER_CHUNK
            pltpu.sync_copy(out_v.at[b], out_hbm.at[pl.ds(f0, OR_PER_CHUNK)])
        return carry

    lax.fori_loop(0, N_CHUNKS // 2, outer, 0)


def kernel(tokens, table):
    table_s = table * jnp.float32(SCALE)
    flat = _emb_lookup(tokens.astype(jnp.int32), table_s)
    one = lax.optimization_barrier(jnp.float32(1.0))
    return flat.reshape(ROWS, COLS, EMB) * one


# speedup vs baseline: 1.0897x; 1.0897x over previous
import functools
import math

import jax
import jax.numpy as jnp
from jax import lax
from jax.experimental import pallas as pl
from jax.experimental.pallas import tpu as pltpu
from jax.experimental.pallas import tpu_sc as plsc

EMB = 32
SCALE = math.sqrt(32.0)

ROWS = 16384
COLS = 50
B = ROWS * COLS
NW = 32
B_PER_W = B // NW
CHUNK = 256
N_CHUNKS = B_PER_W // CHUNK
TOKW = 128
TROWS_PER_CHUNK = CHUNK // TOKW
PACK = TOKW // EMB
OUT_ROWS = B // PACK
OR_PER_CHUNK = CHUNK // PACK
OR_PER_W = B_PER_W // PACK
TABLE_ROWS = 250000

_mesh = plsc.VectorSubcoreMesh(core_axis_name="c", subcore_axis_name="s")


@functools.partial(
    pl.kernel,
    mesh=_mesh,
    out_type=jax.ShapeDtypeStruct((OUT_ROWS, TOKW), jnp.float32),
    scratch_types=[
        pltpu.VMEM((TROWS_PER_CHUNK, TOKW), jnp.int32),
        pltpu.VMEM((TROWS_PER_CHUNK, TOKW), jnp.int32),
        pltpu.VMEM((CHUNK,), jnp.int32),
        pltpu.VMEM((CHUNK,), jnp.int32),
        pltpu.VMEM((CHUNK,), jnp.int32),
        pltpu.VMEM((CHUNK,), jnp.int32),
        pltpu.VMEM((CHUNK, TOKW), jnp.float32),
        pltpu.VMEM((CHUNK, TOKW), jnp.float32),
        pltpu.VMEM((OR_PER_CHUNK, TOKW), jnp.float32),
        pltpu.VMEM((OR_PER_CHUNK, TOKW), jnp.float32),
        pltpu.SemaphoreType.DMA((2,)),
    ],
    compiler_params=pltpu.CompilerParams(use_tc_tiling_on_sc=True),
)
def _emb_lookup(tok_hbm, table_hbm, out_hbm, idx_v0, idx_v1, idxp_v0,
                idxp_v1, q3o_v0, q3o_v1, gath_v0, gath_v1, out_v0, out_v1,
                gsem):
    idx_vs = (idx_v0, idx_v1)
    idxp_vs = (idxp_v0, idxp_v1)
    q3o_vs = (q3o_v0, q3o_v1)
    gath_vs = (gath_v0, gath_v1)
    out_vs = (out_v0, out_v1)
    wid = lax.axis_index("s") * 2 + lax.axis_index("c")
    tbase = wid * (B_PER_W // TOKW)

    def fire(ci, b):
        idx_v, idxp_v, q3o_v, gath_v = (
            idx_vs[b], idxp_vs[b], q3o_vs[b], gath_vs[b])
        r0 = tbase + ci * TROWS_PER_CHUNK
        pltpu.sync_copy(tok_hbm.at[pl.ds(r0, TROWS_PER_CHUNK)], idx_v)

        @plsc.parallel_loop(0, CHUNK // 16, step=1, unroll=4)
        def _(k):
            r = k // (TOKW // 16)
            c = (k % (TOKW // 16)) * 16
            t = idx_v[r, pl.ds(c, 16)]
            idxp_v[pl.ds(k * 16, 16)] = t >> 2
            q3o_v[pl.ds(k * 16, 16)] = (t & 3) * EMB

        pltpu.async_copy(table_hbm.at[idxp_vs[b]], gath_v, gsem.at[b])

    def drain(b):
        pltpu.make_async_copy(
            table_hbm.at[idxp_vs[b]], gath_vs[b], gsem.at[b]
        ).wait()

    fire(0, 0)

    def outer(g, carry):
        for b in (0, 1):
            ci = g * 2 + b
            nb = 1 - b

            @pl.when(ci + 1 < N_CHUNKS)
            def _():
                fire(ci + 1, nb)

            drain(b)

            q3o_v, gath_v, out_v = q3o_vs[b], gath_vs[b], out_vs[b]

            def extract(k0, carry2):
                q16 = q3o_v[pl.ds(k0 * 16, 16)]
                for j in range(16):
                    q3 = q16[j]
                    fr = k0 * PACK + j // PACK
                    c0 = (j % PACK) * EMB
                    kk = k0 * 16 + j
                    out_v[fr, pl.ds(c0, 16)] = (
                        gath_v[kk, pl.ds(q3, 16)] * SCALE
                    )
                    out_v[fr, pl.ds(c0 + 16, 16)] = (
                        gath_v[kk, pl.ds(q3 + 16, 16)] * SCALE
                    )
                return carry2

            lax.fori_loop(0, CHUNK // 16, extract, 0)

            f0 = wid * OR_PER_W + ci * OR_PER_CHUNK
            pltpu.sync_copy(out_v, out_hbm.at[pl.ds(f0, OR_PER_CHUNK)])
        return carry

    lax.fori_loop(0, N_CHUNKS // 2, outer, 0)


def kernel(tokens, table):
    zero = lax.optimization_barrier(jnp.int32(0))
    tok128 = (tokens.astype(jnp.int32) + zero).reshape(B // TOKW, TOKW)
    table128 = table.reshape(TABLE_ROWS, TOKW)
    flat = _emb_lookup(tok128, table128)
    return flat.reshape(ROWS, COLS, EMB)


# speedup vs baseline: 1.3688x; 1.2561x over previous
import functools
import math

import jax
import jax.numpy as jnp
from jax import lax
from jax.experimental import pallas as pl
from jax.experimental.pallas import tpu as pltpu
from jax.experimental.pallas import tpu_sc as plsc

EMB = 32
SCALE = math.sqrt(32.0)

ROWS = 16384
COLS = 50
NW = 32
R_PER_W = ROWS // NW
TR = 16
N_CHUNKS = R_PER_W // TR
LOOK = TR * COLS
OUT_W = 128
PACK = OUT_W // EMB
OUT_ROWS = ROWS * COLS // PACK
OR_PER_CHUNK = LOOK // PACK

_mesh = plsc.VectorSubcoreMesh(core_axis_name="c", subcore_axis_name="s")


@functools.partial(
    pl.kernel,
    mesh=_mesh,
    out_type=jax.ShapeDtypeStruct((OUT_ROWS, OUT_W), jnp.float32),
    scratch_types=[
        pltpu.VMEM((2, TR, COLS), jnp.int32),
        pltpu.VMEM((2, LOOK, EMB), jnp.float32),
        pltpu.VMEM((2, OR_PER_CHUNK, OUT_W), jnp.float32),
        pltpu.SemaphoreType.DMA((2,)),
    ],
    compiler_params=pltpu.CompilerParams(use_tc_tiling_on_sc=False),
)
def _emb_lookup(tok_hbm, table_hbm, out_hbm, idx_v, rows_v, out_v, gsem):
    wid = lax.axis_index("s") * 2 + lax.axis_index("c")
    base = wid * R_PER_W

    def fire(ci, b):
        r0 = base + ci * TR
        pltpu.sync_copy(tok_hbm.at[pl.ds(r0, TR)], idx_v.at[b])
        for r in range(TR):
            pltpu.async_copy(
                table_hbm.at[idx_v.at[b, r]],
                rows_v.at[b, pl.ds(r * COLS, COLS)],
                gsem.at[b],
            )

    def drain(b):
        for r in range(TR):
            pltpu.make_async_copy(
                table_hbm.at[idx_v.at[b, r]],
                rows_v.at[b, pl.ds(r * COLS, COLS)],
                gsem.at[b],
            ).wait()

    fire(0, 0)

    def outer(g, carry):
        for b in (0, 1):
            ci = g * 2 + b
            nb = 1 - b

            @pl.when(ci + 1 < N_CHUNKS)
            def _():
                fire(ci + 1, nb)

            drain(b)

            @plsc.parallel_loop(0, OR_PER_CHUNK, step=1, unroll=4)
            def _(fr):
                for j in range(PACK):
                    for h in (0, 16):
                        out_v[b, fr, pl.ds(j * EMB + h, 16)] = (
                            rows_v[b, fr * PACK + j, pl.ds(h, 16)] * SCALE
                        )

            f0 = wid * (R_PER_W * COLS // PACK) + ci * OR_PER_CHUNK
            pltpu.sync_copy(out_v.at[b], out_hbm.at[pl.ds(f0, OR_PER_CHUNK)])
        return carry

    lax.fori_loop(0, N_CHUNKS // 2, outer, 0)


def kernel(tokens, table):
    out = _emb_lookup(tokens.astype(jnp.int32), table)
    return out.reshape(ROWS, COLS, EMB)
